# baseline (speedup 1.0000x reference)
import functools

import jax
import jax.numpy as jnp
from jax import lax
from jax.experimental import pallas as pl
from jax.experimental.pallas import tpu as pltpu
from jax.experimental.pallas import tpu_sc as plsc

_N = 10000
_E = 320000
_D = 128
_R = 7
_O = 128
_K = 7

_NC = 2
_NS = 16
_NW = _NC * _NS
_EW = _E // _NW
_KC = 80
_NCHUNK = _EW // _KC
_NP = 10240
_RT = _NP // _NS
_RB = _KC
_NB = 4
_NQUAD = 31


def _matmul_y(x, wt3):
    bn = 5000

    def body(x_ref, w_ref, o_ref):
        for k in range(_K):
            o_ref[k] = jnp.dot(x_ref[...], w_ref[k],
                               preferred_element_type=jnp.float32)

    return pl.pallas_call(
        body,
        grid=(_N // bn,),
        in_specs=[
            pl.BlockSpec((bn, _D), lambda i: (i, 0)),
            pl.BlockSpec((_K, _D, _O), lambda i: (0, 0, 0)),
        ],
        out_specs=pl.BlockSpec((_K, bn, _O), lambda i: (0, i, 0)),
        out_shape=jax.ShapeDtypeStruct((_K, _N, _O), jnp.float32),
    )(x, wt3)


def _make_sc_scatter():
    mesh = plsc.VectorSubcoreMesh(core_axis_name="c", subcore_axis_name="s")

    @functools.partial(
        pl.kernel,
        mesh=mesh,
        out_type=jax.ShapeDtypeStruct((_NC, _NP, _O), jnp.float32),
        scratch_types=(
            [
                pltpu.VMEM((_NB, _KC), jnp.int32),
                pltpu.VMEM((_NB, _KC), jnp.int32),
                pltpu.VMEM((_NB, _KC), jnp.int32),
                pltpu.VMEM((_NB, _KC), jnp.float32),
                pltpu.VMEM((_NB, _KC), jnp.int32),
                pltpu.VMEM((_NB, _KC, _O), jnp.float32),
                pltpu.VMEM_SHARED((_NP, _O), jnp.float32),
            ]
            + [pltpu.SemaphoreType.DMA] * (4 * _NB)
        ),
    )
    def sc_scatter(ni_hbm, rel_hbm, no_hbm, ew_hbm, tbl_hbm, out_hbm,
                   ni_v, rel_v, nc_v, ew_v, gc_v, rows_v, acc, *sems):
        c = lax.axis_index("c")
        s = lax.axis_index("s")
        w = s * _NC + c
        sm = sems[0:_NB]
        sn = sems[_NB:2 * _NB]
        sg = sems[2 * _NB:3 * _NB]
        ss = sems[3 * _NB:4 * _NB]

        zv = jnp.zeros((16,), jnp.float32)

        def zbody(i, carry):
            for j in range(_O // 16):
                rows_v[0, i, pl.ds(j * 16, 16)] = zv
            return carry

        lax.fori_loop(0, _RB, zbody, 0)
        zhs = [
            pltpu.async_copy(
                rows_v.at[0], acc.at[pl.ds(s * _RT + t * _RB, _RB)], sems[0])
            for t in range(_RT // _RB)
        ]
        for h in zhs:
            h.wait()
        plsc.subcore_barrier()

        base = w * _EW
        gdn = lax.GatherDimensionNumbers(
            offset_dims=(), collapsed_slice_dims=(0,), start_index_map=(0,))

        def meta_start(ck, b):
            eb = base + ck * _KC
            pltpu.async_copy(ni_hbm.at[pl.ds(eb, _KC)], ni_v.at[b], sm[b])
            pltpu.async_copy(rel_hbm.at[pl.ds(eb, _KC)], rel_v.at[b], sm[b])
            pltpu.async_copy(ew_hbm.at[pl.ds(eb, _KC)], ew_v.at[b], sm[b])

        def meta_wait(ck, b):
            eb = base + ck * _KC
            pltpu.make_async_copy(
                ni_hbm.at[pl.ds(eb, _KC)], ni_v.at[b], sm[b]).wait()
            pltpu.make_async_copy(
                rel_hbm.at[pl.ds(eb, _KC)], rel_v.at[b], sm[b]).wait()
            pltpu.make_async_copy(
                ew_hbm.at[pl.ds(eb, _KC)], ew_v.at[b], sm[b]).wait()

        def gc_build(b):
            for g in range(_KC // 16):
                dl = pl.ds(g * 16, 16)
                gc_v[b, dl] = rel_v[b, dl] * _N + ni_v[b, dl]

        def scale(b):
            def gbody(g, carry):
                ewg = ew_v[b, pl.ds(g * 16, 16)]
                for l in range(16):
                    e = g * 16 + l
                    sp = lax.gather(
                        ewg, jnp.full((16, 1), l, jnp.int32), gdn,
                        slice_sizes=(1,),
                        mode=lax.GatherScatterMode.PROMISE_IN_BOUNDS)
                    for j in range(_O // 16):
                        sl2 = pl.ds(j * 16, 16)
                        rows_v[b, e, sl2] = rows_v[b, e, sl2] * sp
                return carry

            lax.fori_loop(0, _KC // 16, gbody, 0)

        for b in range(_NB):
            meta_start(jnp.int32(b), b)

        def quad_body(k, carry):
            for b in range(_NB):
                ck = k * _NB + b
                meta_wait(ck, b)
                gc_build(b)

                @pl.when(k > 0)
                def _():
                    pltpu.make_async_copy(
                        rows_v.at[b], acc.at[nc_v.at[b]], ss[b]).wait()

                eb = base + ck * _KC
                pltpu.async_copy(no_hbm.at[pl.ds(eb, _KC)], nc_v.at[b], sn[b])
                pltpu.async_copy(tbl_hbm.at[gc_v.at[b]], rows_v.at[b], sg[b])
            for b in range(_NB):
                ck = k * _NB + b
                pltpu.make_async_copy(
                    tbl_hbm.at[gc_v.at[b]], rows_v.at[b], sg[b]).wait()
                scale(b)
                eb = base + ck * _KC
                pltpu.make_async_copy(
                    no_hbm.at[pl.ds(eb, _KC)], nc_v.at[b], sn[b]).wait()
                pltpu.async_copy(
                    rows_v.at[b], acc.at[nc_v.at[b]], ss[b], add=True)

                @pl.when(k < _NQUAD - 1)
                def _():
                    meta_start(ck + _NB, b)
            return carry

        lax.fori_loop(0, _NQUAD, quad_body, 0)

        for b in range(_NB):
            pltpu.make_async_copy(
                rows_v.at[b], acc.at[nc_v.at[b]], ss[b]).wait()
        eb = base + (_NCHUNK - 1) * _KC
        pltpu.sync_copy(ni_hbm.at[pl.ds(eb, _KC)], ni_v.at[0])
        pltpu.sync_copy(rel_hbm.at[pl.ds(eb, _KC)], rel_v.at[0])
        pltpu.sync_copy(no_hbm.at[pl.ds(eb, _KC)], nc_v.at[0])
        pltpu.sync_copy(ew_hbm.at[pl.ds(eb, _KC)], ew_v.at[0])
        gc_build(0)
        pltpu.async_copy(tbl_hbm.at[gc_v.at[0]], rows_v.at[0], sg[0]).wait()
        scale(0)
        pltpu.sync_copy(rows_v.at[0], acc.at[nc_v.at[0]], add=True)

        plsc.subcore_barrier()
        whs = [None] * (_RT // _RB)
        for t in range(_RT // _RB):
            b = t % _NB
            if t >= _NB:
                whs[t - _NB].wait()
            r0 = s * _RT + t * _RB
            pltpu.async_copy(acc.at[pl.ds(r0, _RB)], rows_v.at[b],
                             sg[b]).wait()
            whs[t] = pltpu.async_copy(
                rows_v.at[b], out_hbm.at[c, pl.ds(r0, _RB)], ss[b])
        for t in range(_RT // _RB - _NB, _RT // _RB):
            whs[t].wait()

    return sc_scatter


_sc_scatter = _make_sc_scatter()


def _finish(psum, x, wlt, bias):
    bn = 5000

    def body(p_ref, x_ref, w_ref, b_ref, o_ref):
        loop = jnp.dot(x_ref[...], w_ref[...],
                       preferred_element_type=jnp.float32)
        o_ref[...] = jnp.maximum(
            p_ref[0] + p_ref[1] + loop + b_ref[...], 0.0)

    return pl.pallas_call(
        body,
        grid=(_N // bn,),
        in_specs=[
            pl.BlockSpec((_NC, bn, _O), lambda i: (0, i, 0)),
            pl.BlockSpec((bn, _D), lambda i: (i, 0)),
            pl.BlockSpec((_D, _O), lambda i: (0, 0)),
            pl.BlockSpec((1, _O), lambda i: (0, 0)),
        ],
        out_specs=pl.BlockSpec((bn, _O), lambda i: (i, 0)),
        out_shape=jax.ShapeDtypeStruct((_N, _O), jnp.float32),
    )(psum, x, wlt, bias)


def kernel(x, node_in, node_out, relation, edge_weight,
           W_lin, b_lin, W_loop, b_loop):
    ni = node_in.astype(jnp.int32)
    no = node_out.astype(jnp.int32)
    rel = relation.astype(jnp.int32)
    ew = edge_weight.astype(jnp.float32)

    wt3 = W_lin.reshape(_O, _R, _D).transpose(1, 2, 0)

    y3 = _matmul_y(x, wt3)
    tbl = y3.reshape(_K * _N, _O)
    psum = _sc_scatter(ni, rel, no, ew, tbl)
    bias = (b_lin + b_loop).reshape(1, _O)
    return _finish(psum, x, W_loop.T, bias)


# speedup vs baseline: 1.0163x; 1.0163x over previous
import functools

import jax
import jax.numpy as jnp
from jax import lax
from jax.experimental import pallas as pl
from jax.experimental.pallas import tpu as pltpu
from jax.experimental.pallas import tpu_sc as plsc

_N = 10000
_E = 320000
_D = 128
_R = 7
_O = 128
_K = 7

_NC = 2
_NS = 16
_NW = _NC * _NS
_EW = _E // _NW
_KC = 80
_NCHUNK = _EW // _KC
_NP = 10240
_RT = _NP // _NS
_RB = _KC
_NB = 4
_NQUAD = 31


def _matmul_y(x, wt3):
    bn = 2000

    def body(x_ref, w_ref, o_ref):
        for k in range(_K):
            o_ref[k] = jnp.dot(x_ref[...], w_ref[k],
                               preferred_element_type=jnp.float32)

    return pl.pallas_call(
        body,
        grid=(_N // bn,),
        in_specs=[
            pl.BlockSpec((bn, _D), lambda i: (i, 0)),
            pl.BlockSpec((_K, _D, _O), lambda i: (0, 0, 0)),
        ],
        out_specs=pl.BlockSpec((_K, bn, _O), lambda i: (0, i, 0)),
        out_shape=jax.ShapeDtypeStruct((_K, _N, _O), jnp.float32),
    )(x, wt3)


def _make_sc_scatter():
    mesh = plsc.VectorSubcoreMesh(core_axis_name="c", subcore_axis_name="s")

    @functools.partial(
        pl.kernel,
        mesh=mesh,
        out_type=jax.ShapeDtypeStruct((_NC, _NP, _O), jnp.float32),
        scratch_types=(
            [
                pltpu.VMEM((_NB, _KC), jnp.int32),
                pltpu.VMEM((_NB, _KC), jnp.int32),
                pltpu.VMEM((_NB, _KC), jnp.int32),
                pltpu.VMEM((_NB, _KC), jnp.float32),
                pltpu.VMEM((_NB, _KC), jnp.int32),
                pltpu.VMEM((_NB, _KC, _O), jnp.float32),
                pltpu.VMEM_SHARED((_NP, _O), jnp.float32),
            ]
            + [pltpu.SemaphoreType.DMA] * (4 * _NB)
        ),
    )
    def sc_scatter(ni_hbm, rel_hbm, no_hbm, ew_hbm, tbl_hbm, out_hbm,
                   ni_v, rel_v, nc_v, ew_v, gc_v, rows_v, acc, *sems):
        c = lax.axis_index("c")
        s = lax.axis_index("s")
        w = s * _NC + c
        sm = sems[0:_NB]
        sn = sems[_NB:2 * _NB]
        sg = sems[2 * _NB:3 * _NB]
        ss = sems[3 * _NB:4 * _NB]

        zv = jnp.zeros((16,), jnp.float32)

        def zbody(i, carry):
            for j in range(_O // 16):
                rows_v[0, i, pl.ds(j * 16, 16)] = zv
            return carry

        lax.fori_loop(0, _RB, zbody, 0)
        zhs = [
            pltpu.async_copy(
                rows_v.at[0], acc.at[pl.ds(s * _RT + t * _RB, _RB)], sems[0])
            for t in range(_RT // _RB)
        ]
        for h in zhs:
            h.wait()
        plsc.subcore_barrier()

        base = w * _EW
        gdn = lax.GatherDimensionNumbers(
            offset_dims=(), collapsed_slice_dims=(0,), start_index_map=(0,))

        def meta_start(ck, b):
            eb = base + ck * _KC
            pltpu.async_copy(ni_hbm.at[pl.ds(eb, _KC)], ni_v.at[b], sm[b])
            pltpu.async_copy(rel_hbm.at[pl.ds(eb, _KC)], rel_v.at[b], sm[b])
            pltpu.async_copy(ew_hbm.at[pl.ds(eb, _KC)], ew_v.at[b], sm[b])

        def meta_wait(ck, b):
            eb = base + ck * _KC
            pltpu.make_async_copy(
                ni_hbm.at[pl.ds(eb, _KC)], ni_v.at[b], sm[b]).wait()
            pltpu.make_async_copy(
                rel_hbm.at[pl.ds(eb, _KC)], rel_v.at[b], sm[b]).wait()
            pltpu.make_async_copy(
                ew_hbm.at[pl.ds(eb, _KC)], ew_v.at[b], sm[b]).wait()

        def gc_build(b):
            for g in range(_KC // 16):
                dl = pl.ds(g * 16, 16)
                gc_v[b, dl] = rel_v[b, dl] * _N + ni_v[b, dl]

        def scale(b):
            def gbody(g, carry):
                ewg = ew_v[b, pl.ds(g * 16, 16)]
                for l in range(16):
                    e = g * 16 + l
                    sp = lax.gather(
                        ewg, jnp.full((16, 1), l, jnp.int32), gdn,
                        slice_sizes=(1,),
                        mode=lax.GatherScatterMode.PROMISE_IN_BOUNDS)
                    for j in range(_O // 16):
                        sl2 = pl.ds(j * 16, 16)
                        rows_v[b, e, sl2] = rows_v[b, e, sl2] * sp
                return carry

            lax.fori_loop(0, _KC // 16, gbody, 0)

        for b in range(_NB):
            meta_start(jnp.int32(b), b)

        def quad_body(k, carry):
            for b in range(_NB):
                ck = k * _NB + b
                meta_wait(ck, b)
                gc_build(b)

                @pl.when(k > 0)
                def _():
                    pltpu.make_async_copy(
                        rows_v.at[b], acc.at[nc_v.at[b]], ss[b]).wait()

                eb = base + ck * _KC
                pltpu.async_copy(no_hbm.at[pl.ds(eb, _KC)], nc_v.at[b], sn[b])
                pltpu.async_copy(tbl_hbm.at[gc_v.at[b]], rows_v.at[b], sg[b])
            for b in range(_NB):
                ck = k * _NB + b
                pltpu.make_async_copy(
                    tbl_hbm.at[gc_v.at[b]], rows_v.at[b], sg[b]).wait()
                scale(b)
                eb = base + ck * _KC
                pltpu.make_async_copy(
                    no_hbm.at[pl.ds(eb, _KC)], nc_v.at[b], sn[b]).wait()
                pltpu.async_copy(
                    rows_v.at[b], acc.at[nc_v.at[b]], ss[b], add=True)

                @pl.when(k < _NQUAD - 1)
                def _():
                    meta_start(ck + _NB, b)
            return carry

        lax.fori_loop(0, _NQUAD, quad_body, 0)

        for b in range(_NB):
            pltpu.make_async_copy(
                rows_v.at[b], acc.at[nc_v.at[b]], ss[b]).wait()
        eb = base + (_NCHUNK - 1) * _KC
        pltpu.sync_copy(ni_hbm.at[pl.ds(eb, _KC)], ni_v.at[0])
        pltpu.sync_copy(rel_hbm.at[pl.ds(eb, _KC)], rel_v.at[0])
        pltpu.sync_copy(no_hbm.at[pl.ds(eb, _KC)], nc_v.at[0])
        pltpu.sync_copy(ew_hbm.at[pl.ds(eb, _KC)], ew_v.at[0])
        gc_build(0)
        pltpu.async_copy(tbl_hbm.at[gc_v.at[0]], rows_v.at[0], sg[0]).wait()
        scale(0)
        pltpu.sync_copy(rows_v.at[0], acc.at[nc_v.at[0]], add=True)

        plsc.subcore_barrier()
        whs = [None] * (_RT // _RB)
        for t in range(_RT // _RB):
            b = t % _NB
            if t >= _NB:
                whs[t - _NB].wait()
            r0 = s * _RT + t * _RB
            pltpu.async_copy(acc.at[pl.ds(r0, _RB)], rows_v.at[b],
                             sg[b]).wait()
            whs[t] = pltpu.async_copy(
                rows_v.at[b], out_hbm.at[c, pl.ds(r0, _RB)], ss[b])
        for t in range(_RT // _RB - _NB, _RT // _RB):
            whs[t].wait()

    return sc_scatter


_sc_scatter = _make_sc_scatter()


def _finish(psum, x, wlt, bias):
    bn = 2000

    def body(p_ref, x_ref, w_ref, b_ref, o_ref):
        loop = jnp.dot(x_ref[...], w_ref[...],
                       preferred_element_type=jnp.float32)
        o_ref[...] = jnp.maximum(
            p_ref[0] + p_ref[1] + loop + b_ref[...], 0.0)

    return pl.pallas_call(
        body,
        grid=(_N // bn,),
        in_specs=[
            pl.BlockSpec((_NC, bn, _O), lambda i: (0, i, 0)),
            pl.BlockSpec((bn, _D), lambda i: (i, 0)),
            pl.BlockSpec((_D, _O), lambda i: (0, 0)),
            pl.BlockSpec((1, _O), lambda i: (0, 0)),
        ],
        out_specs=pl.BlockSpec((bn, _O), lambda i: (i, 0)),
        out_shape=jax.ShapeDtypeStruct((_N, _O), jnp.float32),
    )(psum, x, wlt, bias)


def kernel(x, node_in, node_out, relation, edge_weight,
           W_lin, b_lin, W_loop, b_loop):
    ni = node_in.astype(jnp.int32)
    no = node_out.astype(jnp.int32)
    rel = relation.astype(jnp.int32)
    ew = edge_weight.astype(jnp.float32)

    wt3 = W_lin.reshape(_O, _R, _D).transpose(1, 2, 0)

    y3 = _matmul_y(x, wt3)
    tbl = y3.reshape(_K * _N, _O)
    psum = _sc_scatter(ni, rel, no, ew, tbl)
    bias = (b_lin + b_loop).reshape(1, _O)
    return _finish(psum, x, W_loop.T, bias)
